# baseline (speedup 1.0000x reference)
import functools

import jax
import jax.numpy as jnp
from jax import lax
from jax.experimental import pallas as pl
from jax.experimental.pallas import tpu as pltpu
from jax.experimental.pallas import tpu_sc as plsc

P = 128
N = 10000
NA = 10240
E = 320000
EP = 327680
NW = 32
EPW = EP // NW
CPW = EP // 128 // 16
CHH = 40
RPS = NA // 16
G = 16
BN = 2048
T = 4


def _segsum_body(mu_hbm, src_hbm, dst_hbm, z_hbm, out_hbm,
                 src_v, dst_v, rows_v, acc_sh, sem):
    sid = lax.axis_index("s")
    lo = sid * RPS
    pltpu.sync_copy(z_hbm.at[pl.ds(lo, RPS)], acc_sh.at[pl.ds(lo, RPS)])
    plsc.subcore_barrier()

    def _phase(pbase):
        pltpu.sync_copy(src_hbm.at[pl.ds(pbase, CHH)], src_v)
        pltpu.sync_copy(dst_hbm.at[pl.ds(pbase, CHH)], dst_v)
        pltpu.async_copy(mu_hbm.at[src_v.at[0]], rows_v.at[0], sem.at[0])

        @pl.loop(0, CHH)
        def _chunk(c):
            b = lax.rem(c, 2)
            nb = 1 - b

            @pl.when(c + 1 < CHH)
            def _():
                pltpu.async_copy(mu_hbm.at[src_v.at[c + 1]], rows_v.at[nb],
                                 sem.at[nb])

            pltpu.make_async_copy(mu_hbm.at[src_v.at[c]], rows_v.at[b],
                                  sem.at[b]).wait()
            pltpu.sync_copy(rows_v.at[b], acc_sh.at[dst_v.at[c]], add=True)

    for ph in range(CPW // CHH):
        _phase(sid * CPW + ph * CHH)

    plsc.subcore_barrier()
    pltpu.sync_copy(acc_sh.at[pl.ds(lo, RPS)], out_hbm.at[pl.ds(lo, RPS)])


@functools.lru_cache(maxsize=1)
def _sc_kernels():
    mesh = plsc.VectorSubcoreMesh(core_axis_name="c", subcore_axis_name="s",
                                  num_cores=1, num_subcores=16)
    segsum = pl.kernel(
        _segsum_body,
        out_type=jax.ShapeDtypeStruct((NA, P), jnp.float32),
        mesh=mesh,
        scratch_types=[
            pltpu.VMEM((CHH, 128), jnp.int32),
            pltpu.VMEM((CHH, 128), jnp.int32),
            pltpu.VMEM((2, 128, P), jnp.float32),
            pltpu.VMEM_SHARED((NA, P), jnp.float32),
            pltpu.SemaphoreType.DMA((2,)),
        ],
    )
    return segsum


HI = NA // 128
BE = 2000


def _trunc(v):
    return v.astype(jnp.bfloat16).astype(jnp.float32)


def _deg_body(w_ref, dst_ref, out_ref):
    d = dst_ref[...]
    hi = d // 128
    lo = d % 128
    oh_hi = (hi == lax.broadcasted_iota(jnp.int32, (1, HI), 1)
             ).astype(jnp.float32)
    oh_lo = (lo == lax.broadcasted_iota(jnp.int32, (1, 128), 1)
             ).astype(jnp.float32)
    w = w_ref[...]
    ap = oh_lo * jnp.maximum(w, 0.0)
    an = oh_lo * jnp.maximum(-w, 0.0)
    dims = (((0,), (0,)), ((), ()))
    pp = lax.dot_general(oh_hi, ap, dims, preferred_element_type=jnp.float32, precision=lax.Precision.HIGHEST)
    pn = lax.dot_general(oh_hi, an, dims, preferred_element_type=jnp.float32, precision=lax.Precision.HIGHEST)

    @pl.when(pl.program_id(0) == 0)
    def _():
        out_ref[...] = jnp.zeros_like(out_ref)

    out_ref[0] += pp
    out_ref[1] += pn


def _dotT(a, b):
    return lax.dot_general(_trunc(a), _trunc(b), (((1,), (1,)), ((), ())),
                           preferred_element_type=jnp.float32,
                           precision=lax.Precision.HIGHEST)


def _parts13(x_blk, deg_ref, w1_ref, w3_ref, w4_ref):
    part1 = x_blk * w1_ref[...][:, 0][None, :]
    w4 = w4_ref[...][:, 0]
    r4p = jnp.maximum(w4, 0.0)
    r4n = jnp.maximum(-w4, 0.0)
    dp = deg_ref[...][:, 0]
    dn = deg_ref[...][:, 1]
    agg_w = dp[:, None] * r4p[None, :] + dn[:, None] * r4n[None, :]
    part3 = _dotT(agg_w, w3_ref[...])
    return part1, part3


def _round0_body(x_ref, deg_ref, w1_ref, w3_ref, w4_ref, mu_ref):
    part1, part3 = _parts13(x_ref[...], deg_ref, w1_ref, w3_ref, w4_ref)
    mu_ref[...] = jnp.maximum(part1 + part3, 0.0)


def _round_body(acc_ref, x_ref, deg_ref, w1_ref, w2_ref, w3_ref, w4_ref, mu_ref):
    p2 = _dotT(acc_ref[...], w2_ref[...])
    part1, part3 = _parts13(x_ref[...], deg_ref, w1_ref, w3_ref, w4_ref)
    mu_ref[...] = jnp.maximum((part1 + p2) + part3, 0.0)


def _pool_body(bid_ref, mu_ref, out_ref):
    oh = (bid_ref[...] == lax.broadcasted_iota(jnp.int32, (1, G), 1)
          ).astype(jnp.float32)
    part = lax.dot_general(oh, mu_ref[...], (((0,), (0,)), ((), ())),
                           preferred_element_type=jnp.float32, precision=lax.Precision.HIGHEST)

    @pl.when(pl.program_id(0) == 0)
    def _():
        out_ref[...] = jnp.zeros_like(out_ref)

    out_ref[...] += part


def _head_body(pool_ref, bid_ref, mu_ref, w6_ref, w7_ref, w5_ref, out_ref):
    gp = _dotT(pool_ref[...], w6_ref[...])
    oh = (bid_ref[...] == lax.broadcasted_iota(jnp.int32, (1, G), 1)
          ).astype(jnp.float32)
    prep = lax.dot_general(oh, gp, (((1,), (0,)), ((), ())),
                           preferred_element_type=jnp.float32,
                           precision=lax.Precision.HIGHEST)
    h2 = _dotT(mu_ref[...], w7_ref[...])
    w5 = _trunc(w5_ref[...])
    va = w5[0, :P]
    vb = w5[0, P:]
    outa = lax.dot_general(_trunc(jnp.maximum(prep, 0.0)), va,
                           (((1,), (0,)), ((), ())),
                           preferred_element_type=jnp.float32,
                           precision=lax.Precision.HIGHEST)
    outb = lax.dot_general(_trunc(jnp.maximum(h2, 0.0)), vb,
                           (((1,), (0,)), ((), ())),
                           preferred_element_type=jnp.float32,
                           precision=lax.Precision.HIGHEST)
    out_ref[...] = (outa + outb)[:, None]


def _full(shape):
    return pl.BlockSpec(shape, lambda i: tuple(0 for _ in shape))


_ROW = lambda c: pl.BlockSpec((BN, c), lambda i: (i, 0))

_deg_call = pl.pallas_call(
    _deg_body,
    grid=(E // BE,),
    in_specs=[pl.BlockSpec((BE, 1), lambda i: (i, 0)),
              pl.BlockSpec((BE, 1), lambda i: (i, 0))],
    out_specs=_full((2, HI, 128)),
    out_shape=jax.ShapeDtypeStruct((2, HI, 128), jnp.float32),
)

_round0_call = pl.pallas_call(
    _round0_body,
    grid=(NA // BN,),
    in_specs=[_ROW(1), _ROW(2),
              _full((P, 1)), _full((P, P)), _full((P, 1))],
    out_specs=_ROW(P),
    out_shape=jax.ShapeDtypeStruct((NA, P), jnp.float32),
)

_round_call = pl.pallas_call(
    _round_body,
    grid=(NA // BN,),
    in_specs=[_ROW(P), _ROW(1), _ROW(2),
              _full((P, 1)), _full((P, P)), _full((P, P)), _full((P, 1))],
    out_specs=_ROW(P),
    out_shape=jax.ShapeDtypeStruct((NA, P), jnp.float32),
)

_pool_call = pl.pallas_call(
    _pool_body,
    grid=(NA // BN,),
    in_specs=[_ROW(1), _ROW(P)],
    out_specs=_full((G, P)),
    out_shape=jax.ShapeDtypeStruct((G, P), jnp.float32),
)

_head_call = pl.pallas_call(
    _head_body,
    grid=(NA // BN,),
    in_specs=[_full((G, P)), _ROW(1), _ROW(P),
              _full((P, P)), _full((P, P)), _full((1, 2 * P))],
    out_specs=_ROW(1),
    out_shape=jax.ShapeDtypeStruct((NA, 1), jnp.float32),
)


def kernel(x, mu, weight, edge_index, batch_ids, W1s, W2s, W3s, W4s, W5, W6, W7):
    src = edge_index[0].astype(jnp.int32)
    dst = edge_index[1].astype(jnp.int32)
    pad_e = EP - E
    src2d = jnp.concatenate([src, jnp.zeros((pad_e,), jnp.int32)]
                            ).reshape(EP // 128, 128)
    dst2d = jnp.concatenate([dst, jnp.full((pad_e,), N, jnp.int32)]
                            ).reshape(EP // 128, 128)
    bid2d = jnp.concatenate([batch_ids.astype(jnp.int32),
                             jnp.full((NA - N,), G, jnp.int32)]).reshape(NA, 1)
    xp = jnp.concatenate([x, jnp.zeros((NA - N, 1), jnp.float32)])
    zeros = jnp.zeros((NA, P), jnp.float32)

    segsum_call = _sc_kernels()
    deg3d = _deg_call(weight, dst.reshape(E, 1))
    deg = deg3d.reshape(2, NA).T
    mu_c = _round0_call(xp, deg, W1s[0], W3s[0], W4s[0])
    for t in range(1, T):
        acc = segsum_call(mu_c, src2d, dst2d, zeros)
        mu_c = _round_call(acc, xp, deg, W1s[t], W2s[t], W3s[t], W4s[t])
    pool = _pool_call(bid2d, mu_c)
    out = _head_call(pool, bid2d, mu_c, W6, W7, W5)
    return out[:N]


# speedup vs baseline: 1.0381x; 1.0381x over previous
import functools

import jax
import jax.numpy as jnp
from jax import lax
from jax.experimental import pallas as pl
from jax.experimental.pallas import tpu as pltpu
from jax.experimental.pallas import tpu_sc as plsc

P = 128
N = 10000
NA = 10240
E = 320000
EP = 327680
NW = 32
EPW = EP // NW
CPW = EP // 128 // 16
CHH = 40
RPS = NA // 16
G = 16
BN = 2048
T = 4


def _segsum_body(mu_hbm, src_hbm, dst_hbm, z_hbm, out_hbm,
                 src_v, dst_v, rows_v, acc_sh, sem):
    cid = lax.axis_index("c")
    sid = lax.axis_index("s")
    lo = sid * RPS

    def _phase(pbase):
        pltpu.sync_copy(src_hbm.at[pl.ds(pbase, CHH)], src_v)
        pltpu.sync_copy(dst_hbm.at[pl.ds(pbase, CHH)], dst_v)
        pltpu.async_copy(mu_hbm.at[src_v.at[0]], rows_v.at[0], sem.at[0])

        @pl.loop(0, CHH)
        def _chunk(c):
            b = lax.rem(c, 2)
            nb = 1 - b

            @pl.when(c + 1 < CHH)
            def _():
                pltpu.async_copy(mu_hbm.at[src_v.at[c + 1]], rows_v.at[nb],
                                 sem.at[nb])

            pltpu.make_async_copy(mu_hbm.at[src_v.at[c]], rows_v.at[b],
                                  sem.at[b]).wait()
            pltpu.sync_copy(rows_v.at[b], acc_sh.at[dst_v.at[c]], add=True)

    @pl.when(cid == 0)
    def _():
        pltpu.sync_copy(z_hbm.at[pl.ds(lo, RPS)], acc_sh.at[pl.ds(lo, RPS)])
        plsc.subcore_barrier()
        for ph in range(CPW // CHH):
            _phase(sid * CPW + ph * CHH)
        plsc.subcore_barrier()
        pltpu.sync_copy(acc_sh.at[pl.ds(lo, RPS)], out_hbm.at[pl.ds(lo, RPS)])


@functools.lru_cache(maxsize=1)
def _sc_kernels():
    mesh = plsc.VectorSubcoreMesh(core_axis_name="c", subcore_axis_name="s",
                                  num_cores=2, num_subcores=16)
    segsum = pl.kernel(
        _segsum_body,
        out_type=jax.ShapeDtypeStruct((NA, P), jnp.float32),
        mesh=mesh,
        scratch_types=[
            pltpu.VMEM((CHH, 128), jnp.int32),
            pltpu.VMEM((CHH, 128), jnp.int32),
            pltpu.VMEM((2, 128, P), jnp.float32),
            pltpu.VMEM_SHARED((NA, P), jnp.float32),
            pltpu.SemaphoreType.DMA((2,)),
        ],
    )
    return segsum


HI = NA // 128
BE = 4000


def _trunc(v):
    return v.astype(jnp.bfloat16).astype(jnp.float32)


def _deg_body(w_ref, dst_ref, out_ref):
    d = dst_ref[...]
    hi = d // 128
    lo = d % 128
    oh_hi = (hi == lax.broadcasted_iota(jnp.int32, (1, HI), 1)
             ).astype(jnp.float32)
    oh_lo = (lo == lax.broadcasted_iota(jnp.int32, (1, 128), 1)
             ).astype(jnp.float32)
    w = w_ref[...]
    apn = jnp.concatenate([oh_lo * jnp.maximum(w, 0.0),
                           oh_lo * jnp.maximum(-w, 0.0)], axis=1)
    dims = (((0,), (0,)), ((), ()))
    acc = jnp.zeros((HI, 2 * 128), jnp.float32)
    for _ in range(3):
        h = _trunc(apn)
        acc += lax.dot_general(oh_hi, h, dims,
                               preferred_element_type=jnp.float32)
        apn = apn - h

    @pl.when(pl.program_id(0) == 0)
    def _():
        out_ref[...] = jnp.zeros_like(out_ref)

    out_ref[0] += acc[:, :128]
    out_ref[1] += acc[:, 128:]


def _dotT(a, b):
    return lax.dot_general(_trunc(a), _trunc(b), (((1,), (1,)), ((), ())),
                           preferred_element_type=jnp.float32,
                           precision=lax.Precision.HIGHEST)


def _parts13(x_blk, deg_ref, w1_ref, w3_ref, w4_ref):
    part1 = x_blk * w1_ref[...][:, 0][None, :]
    w4 = w4_ref[...][:, 0]
    r4p = jnp.maximum(w4, 0.0)
    r4n = jnp.maximum(-w4, 0.0)
    dp = deg_ref[...][:, 0]
    dn = deg_ref[...][:, 1]
    agg_w = dp[:, None] * r4p[None, :] + dn[:, None] * r4n[None, :]
    part3 = _dotT(agg_w, w3_ref[...])
    return part1, part3


def _round0_body(x_ref, deg_ref, w1_ref, w3_ref, w4_ref, mu_ref):
    part1, part3 = _parts13(x_ref[...], deg_ref, w1_ref, w3_ref, w4_ref)
    mu_ref[...] = jnp.maximum(part1 + part3, 0.0)


def _round_body(acc_ref, x_ref, deg_ref, w1_ref, w2_ref, w3_ref, w4_ref, mu_ref):
    p2 = _dotT(acc_ref[...], w2_ref[...])
    part1, part3 = _parts13(x_ref[...], deg_ref, w1_ref, w3_ref, w4_ref)
    mu_ref[...] = jnp.maximum((part1 + p2) + part3, 0.0)


def _pool_body(bid_ref, mu_ref, out_ref):
    oh = (bid_ref[...] == lax.broadcasted_iota(jnp.int32, (1, G), 1)
          ).astype(jnp.float32)
    part = lax.dot_general(oh, mu_ref[...], (((0,), (0,)), ((), ())),
                           preferred_element_type=jnp.float32, precision=lax.Precision.HIGHEST)

    @pl.when(pl.program_id(0) == 0)
    def _():
        out_ref[...] = jnp.zeros_like(out_ref)

    out_ref[...] += part


def _head_body(pool_ref, bid_ref, mu_ref, w6_ref, w7_ref, w5_ref, out_ref):
    gp = _dotT(pool_ref[...], w6_ref[...])
    oh = (bid_ref[...] == lax.broadcasted_iota(jnp.int32, (1, G), 1)
          ).astype(jnp.float32)
    prep = lax.dot_general(oh, gp, (((1,), (0,)), ((), ())),
                           preferred_element_type=jnp.float32,
                           precision=lax.Precision.HIGHEST)
    h2 = _dotT(mu_ref[...], w7_ref[...])
    w5 = _trunc(w5_ref[...])
    va = w5[0, :P]
    vb = w5[0, P:]
    outa = lax.dot_general(_trunc(jnp.maximum(prep, 0.0)), va,
                           (((1,), (0,)), ((), ())),
                           preferred_element_type=jnp.float32,
                           precision=lax.Precision.HIGHEST)
    outb = lax.dot_general(_trunc(jnp.maximum(h2, 0.0)), vb,
                           (((1,), (0,)), ((), ())),
                           preferred_element_type=jnp.float32,
                           precision=lax.Precision.HIGHEST)
    out_ref[...] = (outa + outb)[:, None]


def _full(shape):
    return pl.BlockSpec(shape, lambda i: tuple(0 for _ in shape))


_ROW = lambda c: pl.BlockSpec((BN, c), lambda i: (i, 0))

_deg_call = pl.pallas_call(
    _deg_body,
    grid=(E // BE,),
    in_specs=[pl.BlockSpec((BE, 1), lambda i: (i, 0)),
              pl.BlockSpec((BE, 1), lambda i: (i, 0))],
    out_specs=_full((2, HI, 128)),
    out_shape=jax.ShapeDtypeStruct((2, HI, 128), jnp.float32),
)

_round0_call = pl.pallas_call(
    _round0_body,
    grid=(NA // BN,),
    in_specs=[_ROW(1), _ROW(2),
              _full((P, 1)), _full((P, P)), _full((P, 1))],
    out_specs=_ROW(P),
    out_shape=jax.ShapeDtypeStruct((NA, P), jnp.float32),
)

_round_call = pl.pallas_call(
    _round_body,
    grid=(NA // BN,),
    in_specs=[_ROW(P), _ROW(1), _ROW(2),
              _full((P, 1)), _full((P, P)), _full((P, P)), _full((P, 1))],
    out_specs=_ROW(P),
    out_shape=jax.ShapeDtypeStruct((NA, P), jnp.float32),
)

_pool_call = pl.pallas_call(
    _pool_body,
    grid=(NA // BN,),
    in_specs=[_ROW(1), _ROW(P)],
    out_specs=_full((G, P)),
    out_shape=jax.ShapeDtypeStruct((G, P), jnp.float32),
)

_head_call = pl.pallas_call(
    _head_body,
    grid=(NA // BN,),
    in_specs=[_full((G, P)), _ROW(1), _ROW(P),
              _full((P, P)), _full((P, P)), _full((1, 2 * P))],
    out_specs=_ROW(1),
    out_shape=jax.ShapeDtypeStruct((NA, 1), jnp.float32),
)


def kernel(x, mu, weight, edge_index, batch_ids, W1s, W2s, W3s, W4s, W5, W6, W7):
    src = edge_index[0].astype(jnp.int32)
    dst = edge_index[1].astype(jnp.int32)
    pad_e = EP - E
    src2d = jnp.concatenate([src, jnp.zeros((pad_e,), jnp.int32)]
                            ).reshape(EP // 128, 128)
    dst2d = jnp.concatenate([dst, jnp.full((pad_e,), N, jnp.int32)]
                            ).reshape(EP // 128, 128)
    bid2d = jnp.concatenate([batch_ids.astype(jnp.int32),
                             jnp.full((NA - N,), G, jnp.int32)]).reshape(NA, 1)
    xp = jnp.concatenate([x, jnp.zeros((NA - N, 1), jnp.float32)])
    zeros = jnp.zeros((NA, P), jnp.float32)

    segsum_call = _sc_kernels()
    deg3d = _deg_call(weight, dst.reshape(E, 1))
    deg = deg3d.reshape(2, NA).T
    mu_c = _round0_call(xp, deg, W1s[0], W3s[0], W4s[0])
    for t in range(1, T):
        acc = segsum_call(mu_c, src2d, dst2d, zeros)
        mu_c = _round_call(acc, xp, deg, W1s[t], W2s[t], W3s[t], W4s[t])
    pool = _pool_call(bid2d, mu_c)
    out = _head_call(pool, bid2d, mu_c, W6, W7, W5)
    return out[:N]


# speedup vs baseline: 1.1392x; 1.0974x over previous
import functools

import jax
import jax.numpy as jnp
from jax import lax
from jax.experimental import pallas as pl
from jax.experimental.pallas import tpu as pltpu
from jax.experimental.pallas import tpu_sc as plsc

P = 128
N = 10000
NA = 10240
E = 320000
EP = 327680
NW = 32
EPW = EP // NW
FCH = 128
SCH = 32
CHH = 32
RPS = NA // 16
G = 16
BN = 2048
T = 4


def _segsum_body(mu_hbm, src_hbm, dst_hbm, z_hbm, out_hbm,
                 src_v, dst_v, rows_v, acc_sh, sem):
    cid = lax.axis_index("c")
    sid = lax.axis_index("s")
    lo = sid * RPS

    def _phase(pbase):
        pltpu.sync_copy(src_hbm.at[pl.ds(pbase, CHH)], src_v)
        pltpu.sync_copy(dst_hbm.at[pl.ds(pbase, CHH)], dst_v)
        pltpu.async_copy(mu_hbm.at[src_v.at[0]], rows_v.at[0], sem.at[0])

        @pl.loop(0, CHH)
        def _chunk(c):
            b = lax.rem(c, 2)
            nb = 1 - b

            @pl.when(c + 1 < CHH)
            def _():
                pltpu.async_copy(mu_hbm.at[src_v.at[c + 1]], rows_v.at[nb],
                                 sem.at[nb])

            pltpu.make_async_copy(mu_hbm.at[src_v.at[c]], rows_v.at[b],
                                  sem.at[b]).wait()
            pltpu.sync_copy(rows_v.at[b], acc_sh.at[dst_v.at[c]], add=True)

    pltpu.sync_copy(z_hbm.at[pl.ds(lo, RPS)], acc_sh.at[pl.ds(lo, RPS)])
    plsc.subcore_barrier()

    base_w = jnp.where(cid == 0, sid * FCH, 16 * FCH + sid * SCH)
    _phase(base_w)
    for ph in range(1, FCH // CHH):
        @pl.when(cid == 0)
        def _():
            _phase(base_w + ph * CHH)

    plsc.subcore_barrier()
    pltpu.sync_copy(acc_sh.at[pl.ds(lo, RPS)], out_hbm.at[cid, pl.ds(lo, RPS)])


@functools.lru_cache(maxsize=1)
def _sc_kernels():
    mesh = plsc.VectorSubcoreMesh(core_axis_name="c", subcore_axis_name="s",
                                  num_cores=2, num_subcores=16)
    segsum = pl.kernel(
        _segsum_body,
        out_type=jax.ShapeDtypeStruct((2, NA, P), jnp.float32),
        mesh=mesh,
        scratch_types=[
            pltpu.VMEM((CHH, 128), jnp.int32),
            pltpu.VMEM((CHH, 128), jnp.int32),
            pltpu.VMEM((2, 128, P), jnp.float32),
            pltpu.VMEM_SHARED((NA, P), jnp.float32),
            pltpu.SemaphoreType.DMA((2,)),
        ],
    )
    return segsum


HI = NA // 128
BE = 4000


def _trunc(v):
    return v.astype(jnp.bfloat16).astype(jnp.float32)


def _deg_body(w_ref, dst_ref, out_ref):
    d = dst_ref[...]
    hi = d // 128
    lo = d % 128
    oh_hi = (hi == lax.broadcasted_iota(jnp.int32, (1, HI), 1)
             ).astype(jnp.float32)
    oh_lo = (lo == lax.broadcasted_iota(jnp.int32, (1, 128), 1)
             ).astype(jnp.float32)
    w = w_ref[...]
    apn = jnp.concatenate([oh_lo * jnp.maximum(w, 0.0),
                           oh_lo * jnp.maximum(-w, 0.0)], axis=1)
    dims = (((0,), (0,)), ((), ()))
    acc = jnp.zeros((HI, 2 * 128), jnp.float32)
    for _ in range(3):
        h = _trunc(apn)
        acc += lax.dot_general(oh_hi, h, dims,
                               preferred_element_type=jnp.float32)
        apn = apn - h

    @pl.when(pl.program_id(0) == 0)
    def _():
        out_ref[...] = jnp.zeros_like(out_ref)

    out_ref[0] += acc[:, :128]
    out_ref[1] += acc[:, 128:]


def _dotT(a, b):
    return lax.dot_general(_trunc(a), _trunc(b), (((1,), (1,)), ((), ())),
                           preferred_element_type=jnp.float32,
                           precision=lax.Precision.HIGHEST)


def _parts13(x_blk, deg_ref, w1_ref, w3_ref, w4_ref):
    part1 = x_blk * w1_ref[...][:, 0][None, :]
    w4 = w4_ref[...][:, 0]
    r4p = jnp.maximum(w4, 0.0)
    r4n = jnp.maximum(-w4, 0.0)
    dp = deg_ref[...][:, 0]
    dn = deg_ref[...][:, 1]
    agg_w = dp[:, None] * r4p[None, :] + dn[:, None] * r4n[None, :]
    part3 = _dotT(agg_w, w3_ref[...])
    return part1, part3


def _round0_body(x_ref, deg_ref, w1_ref, w3_ref, w4_ref, mu_ref):
    part1, part3 = _parts13(x_ref[...], deg_ref, w1_ref, w3_ref, w4_ref)
    mu_ref[...] = jnp.maximum(part1 + part3, 0.0)


def _round_body(acc_ref, x_ref, deg_ref, w1_ref, w2_ref, w3_ref, w4_ref, mu_ref):
    p2 = _dotT(acc_ref[0] + acc_ref[1], w2_ref[...])
    part1, part3 = _parts13(x_ref[...], deg_ref, w1_ref, w3_ref, w4_ref)
    mu_ref[...] = jnp.maximum((part1 + p2) + part3, 0.0)


def _pool_body(bid_ref, mu_ref, out_ref):
    oh = (bid_ref[...] == lax.broadcasted_iota(jnp.int32, (1, G), 1)
          ).astype(jnp.float32)
    part = lax.dot_general(oh, mu_ref[...], (((0,), (0,)), ((), ())),
                           preferred_element_type=jnp.float32, precision=lax.Precision.HIGHEST)

    @pl.when(pl.program_id(0) == 0)
    def _():
        out_ref[...] = jnp.zeros_like(out_ref)

    out_ref[...] += part


def _head_body(pool_ref, bid_ref, mu_ref, w6_ref, w7_ref, w5_ref, out_ref):
    gp = _dotT(pool_ref[...], w6_ref[...])
    oh = (bid_ref[...] == lax.broadcasted_iota(jnp.int32, (1, G), 1)
          ).astype(jnp.float32)
    prep = lax.dot_general(oh, gp, (((1,), (0,)), ((), ())),
                           preferred_element_type=jnp.float32,
                           precision=lax.Precision.HIGHEST)
    h2 = _dotT(mu_ref[...], w7_ref[...])
    w5 = _trunc(w5_ref[...])
    va = w5[0, :P]
    vb = w5[0, P:]
    outa = lax.dot_general(_trunc(jnp.maximum(prep, 0.0)), va,
                           (((1,), (0,)), ((), ())),
                           preferred_element_type=jnp.float32,
                           precision=lax.Precision.HIGHEST)
    outb = lax.dot_general(_trunc(jnp.maximum(h2, 0.0)), vb,
                           (((1,), (0,)), ((), ())),
                           preferred_element_type=jnp.float32,
                           precision=lax.Precision.HIGHEST)
    out_ref[...] = (outa + outb)[:, None]


def _full(shape):
    return pl.BlockSpec(shape, lambda i: tuple(0 for _ in shape))


_ROW = lambda c: pl.BlockSpec((BN, c), lambda i: (i, 0))

_deg_call = pl.pallas_call(
    _deg_body,
    grid=(E // BE,),
    in_specs=[pl.BlockSpec((BE, 1), lambda i: (i, 0)),
              pl.BlockSpec((BE, 1), lambda i: (i, 0))],
    out_specs=_full((2, HI, 128)),
    out_shape=jax.ShapeDtypeStruct((2, HI, 128), jnp.float32),
)

_round0_call = pl.pallas_call(
    _round0_body,
    grid=(NA // BN,),
    in_specs=[_ROW(1), _ROW(2),
              _full((P, 1)), _full((P, P)), _full((P, 1))],
    out_specs=_ROW(P),
    out_shape=jax.ShapeDtypeStruct((NA, P), jnp.float32),
)

_round_call = pl.pallas_call(
    _round_body,
    grid=(NA // BN,),
    in_specs=[pl.BlockSpec((2, BN, P), lambda i: (0, i, 0)), _ROW(1), _ROW(2),
              _full((P, 1)), _full((P, P)), _full((P, P)), _full((P, 1))],
    out_specs=_ROW(P),
    out_shape=jax.ShapeDtypeStruct((NA, P), jnp.float32),
)

_pool_call = pl.pallas_call(
    _pool_body,
    grid=(NA // BN,),
    in_specs=[_ROW(1), _ROW(P)],
    out_specs=_full((G, P)),
    out_shape=jax.ShapeDtypeStruct((G, P), jnp.float32),
)

_head_call = pl.pallas_call(
    _head_body,
    grid=(NA // BN,),
    in_specs=[_full((G, P)), _ROW(1), _ROW(P),
              _full((P, P)), _full((P, P)), _full((1, 2 * P))],
    out_specs=_ROW(1),
    out_shape=jax.ShapeDtypeStruct((NA, 1), jnp.float32),
)


def kernel(x, mu, weight, edge_index, batch_ids, W1s, W2s, W3s, W4s, W5, W6, W7):
    src = edge_index[0].astype(jnp.int32)
    dst = edge_index[1].astype(jnp.int32)
    pad_e = EP - E
    src2d = jnp.concatenate([src, jnp.zeros((pad_e,), jnp.int32)]
                            ).reshape(EP // 128, 128)
    dst2d = jnp.concatenate([dst, jnp.full((pad_e,), N, jnp.int32)]
                            ).reshape(EP // 128, 128)
    bid2d = jnp.concatenate([batch_ids.astype(jnp.int32),
                             jnp.full((NA - N,), G, jnp.int32)]).reshape(NA, 1)
    xp = jnp.concatenate([x, jnp.zeros((NA - N, 1), jnp.float32)])
    zeros = jnp.zeros((NA, P), jnp.float32)

    segsum_call = _sc_kernels()
    deg3d = _deg_call(weight, dst.reshape(E, 1))
    deg = deg3d.reshape(2, NA).T
    mu_c = _round0_call(xp, deg, W1s[0], W3s[0], W4s[0])
    for t in range(1, T):
        acc = segsum_call(mu_c, src2d, dst2d, zeros)
        mu_c = _round_call(acc, xp, deg, W1s[t], W2s[t], W3s[t], W4s[t])
    pool = _pool_call(bid2d, mu_c)
    out = _head_call(pool, bid2d, mu_c, W6, W7, W5)
    return out[:N]
